# Initial kernel scaffold; baseline (speedup 1.0000x reference)
import jax
import jax.numpy as jnp
from jax.experimental import pallas as pl


def kernel(x, mask, token_type_ids, only_key_optim, values, codebook):
    raise NotImplementedError("write your pallas kernel here")


# speedup vs baseline: 1.0104x; 1.0104x over previous
import functools

import jax
import jax.numpy as jnp
from jax import lax
from jax.experimental import pallas as pl
from jax.experimental.pallas import tpu as pltpu
from jax.experimental.pallas import tpu_sc as plsc

_B, _N, _H, _K, _D = 8, 1024, 8, 8192, 32
_BN = _B * _N

_TN = 512
_TK = 2048
_NBN = _BN // _TN
_NK = _K // _TK


def _argmin_body(x_ref, cb_ref, xsq_ref, esq_ref, out_ref, m_scr, i_scr):
    h = pl.program_id(0)
    j = pl.program_id(2)
    x = x_ref[0]
    cb = cb_ref[0]
    dots = lax.dot_general(x.astype(jnp.bfloat16).astype(jnp.float32), cb,
                           (((1,), (1,)), ((), ())),
                           preferred_element_type=jnp.float32)
    score = (xsq_ref[0] - 2.0 * dots) + esq_ref[0]
    lmin = jnp.min(score, axis=1, keepdims=True)
    kidx = lax.broadcasted_iota(jnp.int32, (_TN, _TK), 1) + j * _TK
    big = jnp.int32(2 ** 30)
    lidx = jnp.min(jnp.where(score == lmin, kidx, big), axis=1, keepdims=True)

    @pl.when(j == 0)
    def _():
        m_scr[...] = lmin
        i_scr[...] = lidx

    @pl.when(j > 0)
    def _():
        prev_m = m_scr[...]
        take = lmin < prev_m
        m_scr[...] = jnp.where(take, lmin, prev_m)
        i_scr[...] = jnp.where(take, lidx, i_scr[...])

    @pl.when(j == pl.num_programs(2) - 1)
    def _():
        out_ref[0] = i_scr[...] + h * _K


def _argmin_indices(xr, codebook, xsq, esq):
    return pl.pallas_call(
        _argmin_body,
        grid=(_H, _NBN, _NK),
        in_specs=[
            pl.BlockSpec((1, _TN, _D), lambda h, i, j: (h, i, 0)),
            pl.BlockSpec((1, _TK, _D), lambda h, i, j: (h, j, 0)),
            pl.BlockSpec((1, _TN, 1), lambda h, i, j: (h * _NBN + i, 0, 0)),
            pl.BlockSpec((1, 1, _TK), lambda h, i, j: (h * _NK + j, 0, 0)),
        ],
        out_specs=pl.BlockSpec((1, _TN, 1), lambda h, i, j: (h * _NBN + i, 0, 0)),
        out_shape=jax.ShapeDtypeStruct((_H * _NBN, _TN, 1), jnp.int32),
        scratch_shapes=[
            pltpu.VMEM((_TN, 1), jnp.float32),
            pltpu.VMEM((_TN, 1), jnp.int32),
        ],
    )(xr, codebook, xsq, esq)


_NC, _NS = 2, 16
_NW = _NC * _NS
_ROWS = _BN * _H
_RPW = _ROWS // _NW
_CH = 128
_NCH = _RPW // _CH


@functools.cache
def _gather_rows_kernel():
    @functools.partial(
        pl.kernel,
        mesh=plsc.VectorSubcoreMesh(core_axis_name="c", subcore_axis_name="s"),
        out_type=jax.ShapeDtypeStruct((_ROWS, _D), jnp.float32),
        scratch_types=[
            pltpu.VMEM((_NCH, _CH), jnp.int32),
            pltpu.VMEM((_RPW, _D), jnp.float32),
            pltpu.SemaphoreType.DMA,
        ],
        compiler_params=pltpu.CompilerParams(use_tc_tiling_on_sc=False),
    )
    def _gather_rows(table_hbm, idx_hbm, out_hbm, idx_v, rows_v, sem):
        wid = lax.axis_index("s") * _NC + lax.axis_index("c")
        pltpu.sync_copy(idx_hbm.at[wid], idx_v)
        cps = [
            pltpu.async_copy(table_hbm.at[idx_v.at[jj]],
                             rows_v.at[pl.ds(jj * _CH, _CH)], sem)
            for jj in range(_NCH)
        ]
        for cp in cps:
            cp.wait()
        pltpu.sync_copy(rows_v, out_hbm.at[pl.ds(wid * _RPW, _RPW)])

    return _gather_rows


def kernel(x, mask, token_type_ids, only_key_optim, values, codebook):
    xr = x.reshape(_BN, _H, _D).transpose(1, 0, 2)
    xsq = jnp.sum(xr * xr, axis=-1).reshape(_H * _NBN, _TN, 1)
    esq = jnp.sum(codebook * codebook, axis=-1).reshape(_H * _NK, 1, _TK)
    idx = _argmin_indices(xr, codebook, xsq, esq)
    idx = idx.reshape(_H, _BN).T.reshape(_NW, _NCH, _CH)
    table = values.reshape(_H * _K, _D)
    rows = _gather_rows_kernel()(table, idx)
    return rows.reshape(_B, _N, _H * _D)
